# baseline (speedup 1.0000x reference)
import functools

import jax
import jax.numpy as jnp
from jax import lax
from jax.experimental import pallas as pl
from jax.experimental.pallas import tpu as pltpu
from jax.experimental.pallas import tpu_sc as plsc


def kernel(input_ids, token_table, position_table):
    B, S = input_ids.shape
    V, D = token_table.shape
    N = B * S
    L = 16

    info = plsc.get_sparse_core_info()
    NC, NS = info.num_cores, info.num_subcores
    NW = NC * NS
    SB = S // NW
    QB = 16
    NQ = SB // QB
    NSEC = 3

    ids_re = (
        input_ids.astype(jnp.int32)
        .reshape(B, NW, NQ, QB)
        .transpose(1, 2, 0, 3)
        .reshape(N)
    )
    mesh = plsc.VectorSubcoreMesh(core_axis_name="c", subcore_axis_name="s")

    @functools.partial(
        pl.kernel,
        mesh=mesh,
        out_type=jax.ShapeDtypeStruct((N, D), jnp.float32),
        scratch_types=[
            pltpu.VMEM((NQ * B * QB,), jnp.int32),
            pltpu.VMEM((QB, D), jnp.float32),
            pltpu.VMEM((QB, D), jnp.float32),
            pltpu.VMEM((B * QB, D), jnp.float32),
            pltpu.VMEM((B * QB, D), jnp.float32),
            pltpu.SemaphoreType.DMA,
            pltpu.SemaphoreType.DMA,
            pltpu.SemaphoreType.DMA,
            pltpu.SemaphoreType.DMA,
            pltpu.SemaphoreType.DMA,
            pltpu.SemaphoreType.DMA,
            pltpu.SemaphoreType.DMA,
        ],
    )
    def emb(ids_hbm, tok_hbm, pos_hbm, out_hbm, idx_v, p0, p1, t0, t1,
            g0, g1, q0, q1, s0_sem, s1_sem, isem):
        pos_bufs = (p0, p1)
        tok_bufs = (t0, t1)
        gsems = (g0, g1)
        psems = (q0, q1)
        ssems = (s0_sem, s1_sem)
        wid = lax.axis_index("s") * NC + lax.axis_index("c")
        s0 = wid * SB


        def chunk_gather(q, buf):
            return pltpu.async_copy(
                tok_hbm.at[idx_v.at[pl.ds(q * B * QB, B * QB)]],
                tok_bufs[buf],
                gsems[buf],
            )

        def chunk_pos(q, buf):
            return pltpu.async_copy(
                pos_hbm.at[pl.ds(s0 + q * QB, QB)], pos_bufs[buf], psems[buf]
            )

        CW = B * QB
        pltpu.sync_copy(ids_hbm.at[pl.ds(wid * NQ * CW, CW)],
                        idx_v.at[pl.ds(0, CW)])
        pos_h = [None, None]
        gather_h = [None, None]
        store_h = [None, None]
        pos_h[0] = chunk_pos(0, 0)
        gather_h[0] = chunk_gather(0, 0)
        idx_rest_h = pltpu.async_copy(
            ids_hbm.at[pl.ds(wid * NQ * CW + CW, (NQ - 1) * CW)],
            idx_v.at[pl.ds(CW, (NQ - 1) * CW)],
            isem,
        )

        for q in range(NQ):
            buf = q % 2
            nb = 1 - buf
            if q + 1 < NQ:
                if idx_rest_h is not None:
                    idx_rest_h.wait()
                    idx_rest_h = None
                if store_h[nb] is not None:
                    for h in store_h[nb]:
                        h.wait()
                    store_h[nb] = None
                gather_h[nb] = chunk_gather(q + 1, nb)
                pos_h[nb] = chunk_pos(q + 1, nb)
            gather_h[buf].wait()
            pos_h[buf].wait()

            tok_v = tok_bufs[buf]
            pos_v = pos_bufs[buf]

            def make_row_add(b_lo, b_hi):
                def row_add(r, carry):
                    for sec in range(NSEC):
                        pvs = [
                            pos_v[r, pl.ds((sec * 16 + j) * L, L)]
                            for j in range(16)
                        ]
                        for b in range(b_lo, b_hi):
                            for j in range(16):
                                col = (sec * 16 + j) * L
                                plsc.addupdate(
                                    tok_v.at[b * QB + r, pl.ds(col, L)],
                                    pvs[j],
                                )
                    return carry
                return row_add

            def b_store(b):
                return pltpu.async_copy(
                    tok_v.at[pl.ds(b * QB, QB)],
                    out_hbm.at[pl.ds(b * S + s0 + q * QB, QB)],
                    ssems[buf],
                )

            if q + 1 < NQ:
                lax.fori_loop(0, QB, make_row_add(0, B), 0)
                store_h[buf] = [b_store(b) for b in range(B)]
            else:
                lax.fori_loop(0, QB, make_row_add(0, B // 2), 0)
                early = [b_store(b) for b in range(B // 2)]
                lax.fori_loop(0, QB, make_row_add(B // 2, B), 0)
                store_h[buf] = early + [b_store(b) for b in range(B // 2, B)]
        for sl in range(2):
            if store_h[sl] is not None:
                for h in store_h[sl]:
                    h.wait()

    out = emb(ids_re, token_table, position_table)
    return out.reshape(B, S, D)


# speedup vs baseline: 1.0021x; 1.0021x over previous
import functools

import jax
import jax.numpy as jnp
from jax import lax
from jax.experimental import pallas as pl
from jax.experimental.pallas import tpu as pltpu
from jax.experimental.pallas import tpu_sc as plsc


def kernel(input_ids, token_table, position_table):
    B, S = input_ids.shape
    V, D = token_table.shape
    N = B * S
    L = 16

    info = plsc.get_sparse_core_info()
    NC, NS = info.num_cores, info.num_subcores
    NW = NC * NS
    SB = S // NW
    QB = 16
    NQ = SB // QB
    NSEC = 3

    ids_re = (
        input_ids.astype(jnp.int32)
        .reshape(B, NW, NQ, QB)
        .transpose(1, 2, 0, 3)
        .reshape(N)
    )
    mesh = plsc.VectorSubcoreMesh(core_axis_name="c", subcore_axis_name="s")

    @functools.partial(
        pl.kernel,
        mesh=mesh,
        out_type=jax.ShapeDtypeStruct((N, D), jnp.float32),
        scratch_types=[
            pltpu.VMEM((NQ * B * QB,), jnp.int32),
            pltpu.VMEM((QB, D), jnp.float32),
            pltpu.VMEM((QB, D), jnp.float32),
            pltpu.VMEM((B * QB, D), jnp.float32),
            pltpu.VMEM((B * QB, D), jnp.float32),
            pltpu.SemaphoreType.DMA,
            pltpu.SemaphoreType.DMA,
            pltpu.SemaphoreType.DMA,
            pltpu.SemaphoreType.DMA,
            pltpu.SemaphoreType.DMA,
            pltpu.SemaphoreType.DMA,
            pltpu.SemaphoreType.DMA,
        ],
    )
    def emb(ids_hbm, tok_hbm, pos_hbm, out_hbm, idx_v, p0, p1, t0, t1,
            g0, g1, q0, q1, s0_sem, s1_sem, isem):
        pos_bufs = (p0, p1)
        tok_bufs = (t0, t1)
        gsems = (g0, g1)
        psems = (q0, q1)
        ssems = (s0_sem, s1_sem)
        wid = lax.axis_index("s") * NC + lax.axis_index("c")
        s0 = wid * SB


        def chunk_gather(q, buf):
            return pltpu.async_copy(
                tok_hbm.at[idx_v.at[pl.ds(q * B * QB, B * QB)]],
                tok_bufs[buf],
                gsems[buf],
            )

        def chunk_pos(q, buf):
            return pltpu.async_copy(
                pos_hbm.at[pl.ds(s0 + q * QB, QB)], pos_bufs[buf], psems[buf]
            )

        CW = B * QB
        pltpu.sync_copy(ids_hbm.at[pl.ds(wid * NQ * CW, CW)],
                        idx_v.at[pl.ds(0, CW)])
        pos_h = [None, None]
        gather_h = [None, None]
        store_h = [None, None]
        pos_h[0] = chunk_pos(0, 0)
        gather_h[0] = chunk_gather(0, 0)
        idx_rest_h = pltpu.async_copy(
            ids_hbm.at[pl.ds(wid * NQ * CW + CW, (NQ - 1) * CW)],
            idx_v.at[pl.ds(CW, (NQ - 1) * CW)],
            isem,
        )

        for q in range(NQ):
            buf = q % 2
            nb = 1 - buf
            if q + 1 < NQ:
                if idx_rest_h is not None:
                    idx_rest_h.wait()
                    idx_rest_h = None
                if store_h[nb] is not None:
                    for h in store_h[nb]:
                        h.wait()
                    store_h[nb] = None
                gather_h[nb] = chunk_gather(q + 1, nb)
                pos_h[nb] = chunk_pos(q + 1, nb)
            gather_h[buf].wait()
            pos_h[buf].wait()

            tok_v = tok_bufs[buf]
            pos_v = pos_bufs[buf]

            def row_add(r, carry):
                for sec in range(NSEC):
                    pvs = [
                        pos_v[r, pl.ds((sec * 16 + j) * L, L)]
                        for j in range(16)
                    ]
                    for b in range(B):
                        for j in range(16):
                            col = (sec * 16 + j) * L
                            plsc.addupdate(
                                tok_v.at[b * QB + r, pl.ds(col, L)], pvs[j]
                            )
                return carry

            lax.fori_loop(0, QB, row_add, 0)
            store_h[buf] = [
                pltpu.async_copy(
                    tok_v.at[pl.ds(b * QB, QB)],
                    out_hbm.at[pl.ds(b * S + s0 + q * QB, QB)],
                    ssems[buf],
                )
                for b in range(B)
            ]
        for sl in range(2):
            if store_h[sl] is not None:
                for h in store_h[sl]:
                    h.wait()

    out = emb(ids_re, token_table, position_table)
    return out.reshape(B, S, D)
